# baseline (speedup 1.0000x reference)
import functools

import jax
import jax.numpy as jnp
from jax import lax
from jax.experimental import pallas as pl
from jax.experimental.pallas import tpu as pltpu
from jax.experimental.pallas import tpu_sc as plsc

NC, NS, L = 2, 16, 16
NW = NC * NS
B = 16384
BPW = B // NW
HALF = BPW // 2
E = 64


def _sc_gather(uidx, iidx, ue_gmf, ie_gmf, ue_mlp, ie_mlp):
    mesh = plsc.VectorSubcoreMesh(
        core_axis_name="c", subcore_axis_name="s", num_cores=NC, num_subcores=NS
    )

    @functools.partial(
        pl.kernel,
        out_type=(
            jax.ShapeDtypeStruct((B, 2 * E), jnp.float32),
            jax.ShapeDtypeStruct((B, 2 * E), jnp.float32),
        ),
        mesh=mesh,
        scratch_types=[
            pltpu.VMEM((BPW,), jnp.int32),
            pltpu.VMEM((BPW,), jnp.int32),
            pltpu.VMEM((HALF, 2 * E), jnp.float32),
            pltpu.VMEM((HALF, 2 * E), jnp.float32),
            pltpu.SemaphoreType.DMA,
        ],
    )
    def k(uidx_hbm, iidx_hbm, ueg, ieg, uem, iem,
          mlp_out, gmf_out, uidx_v, iidx_v, mlp_v, gmf_v, sem):
        wid = lax.axis_index("s") * NC + lax.axis_index("c")
        base = wid * BPW

        pltpu.sync_copy(uidx_hbm.at[pl.ds(base, BPW)], uidx_v)
        pltpu.sync_copy(iidx_hbm.at[pl.ds(base, BPW)], iidx_v)

        for h in range(2):
            off = h * HALF

            def body(j, _):
                uvec = uidx_v[pl.ds(off + 16 * j, 16)]
                ivec = iidx_v[pl.ds(off + 16 * j, 16)]
                for kk in range(16):
                    u = uvec[kk]
                    i = ivec[kk]
                    d = 16 * j + kk
                    pltpu.async_copy(uem.at[u], mlp_v.at[d, pl.ds(0, E)], sem)
                    pltpu.async_copy(iem.at[i], mlp_v.at[d, pl.ds(E, E)], sem)
                    pltpu.async_copy(ueg.at[u], gmf_v.at[d, pl.ds(0, E)], sem)
                    pltpu.async_copy(ieg.at[i], gmf_v.at[d, pl.ds(E, E)], sem)
                return 0

            lax.fori_loop(0, HALF // 16, body, 0)
            pltpu.make_async_copy(mlp_out.at[pl.ds(0, HALF)], mlp_v, sem).wait()
            pltpu.make_async_copy(gmf_out.at[pl.ds(0, HALF)], gmf_v, sem).wait()
            pltpu.sync_copy(mlp_v, mlp_out.at[pl.ds(base + off, HALF)])
            pltpu.sync_copy(gmf_v, gmf_out.at[pl.ds(base + off, HALF)])

    return k(uidx, iidx, ue_gmf, ie_gmf, ue_mlp, ie_mlp)


BLK = 2048


def _tc_tower(mlp_in, gmf_in, w1t, b1r, w2t, b2r, w3t, b3r, wpg_r, wpm_r, bp_r):
    def body(x_ref, g_ref, w1_ref, b1_ref, w2_ref, b2_ref, w3_ref, b3_ref,
             wpg_ref, wpm_ref, bp_ref, o_ref):
        g = g_ref[...]
        gdot = jnp.sum(g[:, :E] * g[:, E:] * wpg_ref[...], axis=1)
        h = jnp.dot(x_ref[...], w1_ref[...], preferred_element_type=jnp.float32)
        h = jnp.maximum(h + b1_ref[...], 0.0)
        h = jnp.maximum(
            jnp.dot(h, w2_ref[...], preferred_element_type=jnp.float32)
            + b2_ref[...], 0.0)
        h = jnp.maximum(
            jnp.dot(h, w3_ref[...], preferred_element_type=jnp.float32)
            + b3_ref[...], 0.0)
        o_ref[...] = gdot + jnp.sum(h * wpm_ref[...], axis=1) + bp_ref[0, 0]

    full = lambda r, c: pl.BlockSpec((r, c), lambda i: (0, 0))
    out = pl.pallas_call(
        body,
        grid=(B // BLK,),
        in_specs=[
            pl.BlockSpec((BLK, 2 * E), lambda i: (i, 0)),
            pl.BlockSpec((BLK, 2 * E), lambda i: (i, 0)),
            full(2 * E, E), full(1, E),
            full(E, 32), full(1, 32),
            full(32, 16), full(1, 16),
            full(1, E), full(1, 16), full(1, 1),
        ],
        out_specs=pl.BlockSpec((BLK,), lambda i: (i,)),
        out_shape=jax.ShapeDtypeStruct((B,), jnp.float32),
    )(mlp_in, gmf_in, w1t, b1r, w2t, b2r, w3t, b3r, wpg_r, wpm_r, bp_r)
    return out


def kernel(user_indices, item_indices, ue_gmf, ie_gmf, ue_mlp, ie_mlp,
           W1, b1, W2, b2, W3, b3, Wp, bp):
    uidx = user_indices.astype(jnp.int32)
    iidx = item_indices.astype(jnp.int32)

    mlp_in, gmf_in = _sc_gather(uidx, iidx, ue_gmf, ie_gmf, ue_mlp, ie_mlp)

    return _tc_tower(mlp_in, gmf_in,
                     W1.T, b1.reshape(1, E),
                     W2.T, b2.reshape(1, 32),
                     W3.T, b3.reshape(1, 16),
                     Wp[0, :E].reshape(1, E), Wp[0, E:].reshape(1, L),
                     bp.reshape(1, 1))
